# baseline (speedup 1.0000x reference)
import functools

import jax
import jax.numpy as jnp
from jax import lax
from jax.experimental import pallas as pl
from jax.experimental.pallas import tpu as pltpu
from jax.experimental.pallas import tpu_sc as plsc

N, D_IN, D_OUT, C = 8192, 512, 512, 64
BIG = 2147483647

NC, NS = 2, 16
NSA = 16
NW = NC * NSA
F_SC = 2048
ROWS_SC_W = F_SC // NW
SCCHUNK = 32
NCHUNK = ROWS_SC_W // SCCHUNK
NV = D_IN // 16

TCCHUNK = 512
TC_K0 = F_SC // TCCHUNK
TC_K = (N - F_SC) // TCCHUNK
FIN_K = 1

_sc_mesh = plsc.VectorSubcoreMesh(core_axis_name="c", subcore_axis_name="s")


@functools.partial(
    pl.kernel,
    out_type=jax.ShapeDtypeStruct((2 * NW, C, D_IN), jnp.float32),
    mesh=_sc_mesh,
    scratch_types=[
        pltpu.VMEM((2, SCCHUNK, D_IN), jnp.float32),
        pltpu.VMEM((ROWS_SC_W + 16,), jnp.int32),
        pltpu.VMEM((C, D_IN), jnp.float32),
        pltpu.VMEM((C, D_IN), jnp.float32),
        pltpu.SemaphoreType.DMA,
        pltpu.SemaphoreType.DMA,
    ],
)
def _sc_segsum(x_hbm, lab_hbm, sums_hbm, xbuf, labv, acc0, acc1, sem0, sem1):
    c = lax.axis_index("c")
    s = lax.axis_index("s")
    wid = c * NSA + s
    base = wid * ROWS_SC_W

    @pl.when(s < NSA)
    def _worker():
        _sc_segsum_worker(x_hbm, lab_hbm, sums_hbm, xbuf, labv, acc0, acc1,
                          sem0, sem1, wid, base)


def _sc_segsum_worker(x_hbm, lab_hbm, sums_hbm, xbuf, labv, acc0, acc1,
                      sem0, sem1, wid, base):

    pltpu.sync_copy(lab_hbm.at[pl.ds(base, ROWS_SC_W)],
                    labv.at[pl.ds(0, ROWS_SC_W)])
    pltpu.async_copy(x_hbm.at[pl.ds(base, SCCHUNK)], xbuf.at[0], sem0)

    zv = jnp.zeros((16,), jnp.float32)

    def zero_row(r, carry):
        for jj in range(NV):
            acc0[r, pl.ds(jj * 16, 16)] = zv
            acc1[r, pl.ds(jj * 16, 16)] = zv
        return carry

    lax.fori_loop(0, C, zero_row, 0)

    def process(buf, chunk):
        def row_group(g, carry):
            labvec = labv[pl.ds(chunk * SCCHUNK + g * 8, 16)]
            LAG = 8
            for r8 in range(8):
                lab = labvec[r8]
                dst = acc0 if r8 % 2 == 0 else acc1
                row = g * 8 + r8
                vals = [None] * NV
                for jj in range(NV + LAG):
                    if jj < NV:
                        vals[jj] = xbuf[buf, row, pl.ds(jj * 16, 16)]
                    if jj >= LAG:
                        plsc.addupdate(
                            dst.at[lab, pl.ds((jj - LAG) * 16, 16)],
                            vals[jj - LAG])
            return carry

        lax.fori_loop(0, SCCHUNK // 8, row_group, 0)

    def pair_body(i2, carry):
        ca = 2 * i2
        pltpu.make_async_copy(
            x_hbm.at[pl.ds(base + ca * SCCHUNK, SCCHUNK)], xbuf.at[0], sem0
        ).wait()
        pltpu.async_copy(
            x_hbm.at[pl.ds(base + (ca + 1) * SCCHUNK, SCCHUNK)], xbuf.at[1],
            sem1)
        process(0, ca)
        pltpu.make_async_copy(
            x_hbm.at[pl.ds(base + (ca + 1) * SCCHUNK, SCCHUNK)], xbuf.at[1],
            sem1).wait()

        @pl.when(i2 + 1 < NCHUNK // 2)
        def _next():
            pltpu.async_copy(
                x_hbm.at[pl.ds(base + (ca + 2) * SCCHUNK, SCCHUNK)],
                xbuf.at[0], sem0)

        process(1, ca + 1)
        return carry

    lax.fori_loop(0, NCHUNK // 2, pair_body, 0)

    pltpu.sync_copy(acc0, sums_hbm.at[2 * wid])
    pltpu.sync_copy(acc1, sums_hbm.at[2 * wid + 1])


def _tc_segsum_body(labels_ref, x_ref, sums_ref, cnt_ref, fi_ref):
    k = pl.program_id(0)

    @pl.when(k == 0)
    def _init():
        sums_ref[...] = jnp.zeros_like(sums_ref)
        cnt_ref[...] = jnp.zeros_like(cnt_ref)
        fi_ref[...] = jnp.full_like(fi_ref, BIG)

    labels = labels_ref[0]
    lab_b = jnp.broadcast_to(labels, (C, TCCHUNK))
    class_ids = lax.broadcasted_iota(jnp.int32, (C, TCCHUNK), 0)
    onehot = lab_b == class_ids
    cnt = jnp.sum(onehot.astype(jnp.float32), axis=1, keepdims=True)
    cnt_ref[...] += jnp.broadcast_to(cnt, (C, 128))
    row_idx = ((k + TC_K0) * TCCHUNK
               + lax.broadcasted_iota(jnp.int32, (C, TCCHUNK), 1))
    masked = jnp.where(onehot, row_idx, BIG)
    fi_ref[...] = jnp.minimum(
        fi_ref[...],
        jnp.broadcast_to(jnp.min(masked, axis=1, keepdims=True), (C, 128)))

    x = x_ref[...]
    x_hi = x.astype(jnp.bfloat16)
    x_lo = (x - x_hi.astype(jnp.float32)).astype(jnp.bfloat16)
    oh_bf = onehot.astype(jnp.bfloat16)
    dn = (((1,), (0,)), ((), ()))
    sums_ref[...] += (
        lax.dot_general(oh_bf, x_hi, dimension_numbers=dn,
                        preferred_element_type=jnp.float32)
        + lax.dot_general(oh_bf, x_lo, dimension_numbers=dn,
                          preferred_element_type=jnp.float32)
    )


def _tc_segsum(labels3d, inputs):
    return pl.pallas_call(
        _tc_segsum_body,
        grid=(TC_K,),
        in_specs=[
            pl.BlockSpec((1, 1, TCCHUNK), lambda k: (k + TC_K0, 0, 0)),
            pl.BlockSpec((TCCHUNK, D_IN), lambda k: (k + TC_K0, 0)),
        ],
        out_specs=[
            pl.BlockSpec((C, D_IN), lambda k: (0, 0)),
            pl.BlockSpec((C, 128), lambda k: (0, 0)),
            pl.BlockSpec((C, 128), lambda k: (0, 0)),
        ],
        out_shape=[
            jax.ShapeDtypeStruct((C, D_IN), jnp.float32),
            jax.ShapeDtypeStruct((C, 128), jnp.float32),
            jax.ShapeDtypeStruct((C, 128), jnp.int32),
        ],
        compiler_params=pltpu.CompilerParams(
            dimension_semantics=("arbitrary",),
        ),
    )(labels3d, inputs)


def _tc_body(psums_ref, tcsums_ref, cnt_ref, fi_ref, labsc_ref, W_ref,
             b_ref, out_ref, acc_ref):
    k = pl.program_id(0)
    PG = 2 * NW // FIN_K

    @pl.when(k == 0)
    def _init():
        acc_ref[...] = tcsums_ref[...]

    sums = psums_ref[0]
    for t in range(1, PG):
        sums = sums + psums_ref[t]
    acc_ref[...] += sums

    @pl.when(k == FIN_K - 1)
    def _finish():
        labels = labsc_ref[:, :F_SC]
        lab_b = jnp.broadcast_to(labels, (C, F_SC))
        class_ids = lax.broadcasted_iota(jnp.int32, (C, F_SC), 0)
        onehot = lab_b == class_ids
        cnt0 = jnp.sum(onehot.astype(jnp.float32), axis=1, keepdims=True)
        row_idx = lax.broadcasted_iota(jnp.int32, (C, F_SC), 1)
        fi0 = jnp.min(jnp.where(onehot, row_idx, BIG), axis=1, keepdims=True)

        cnt_col = cnt_ref[:, :1] + cnt0
        fi_col_i = jnp.minimum(fi_ref[:, :1], fi0)

        sums_all = acc_ref[...]
        means = sums_all / jnp.broadcast_to(cnt_col, (C, D_IN))
        proto = lax.dot_general(
            means, W_ref[...], dimension_numbers=(((1,), (0,)), ((), ())),
            preferred_element_type=jnp.float32,
            precision=lax.Precision.HIGHEST,
        ) + b_ref[...]

        fi_col = fi_col_i.astype(jnp.float32)
        eye = (lax.broadcasted_iota(jnp.int32, (C, C), 0)
               == lax.broadcasted_iota(jnp.int32, (C, C), 1)
               ).astype(jnp.float32)
        fi_rowv = lax.dot_general(
            fi_col, eye, dimension_numbers=(((0,), (0,)), ((), ())),
            preferred_element_type=jnp.float32,
            precision=lax.Precision.HIGHEST,
        )
        fi_lanes = jnp.broadcast_to(fi_rowv, (C, C))
        fi_subl = jnp.broadcast_to(fi_col, (C, C))
        lane_id = lax.broadcasted_iota(jnp.int32, (C, C), 1)
        subl_id = lax.broadcasted_iota(jnp.int32, (C, C), 0)
        less = ((fi_lanes < fi_subl)
                | ((fi_lanes == fi_subl) & (lane_id < subl_id)))
        rank_col = jnp.sum(less.astype(jnp.float32), axis=1, keepdims=True)
        rank_rowv = lax.dot_general(
            rank_col, eye, dimension_numbers=(((0,), (0,)), ((), ())),
            preferred_element_type=jnp.float32,
            precision=lax.Precision.HIGHEST,
        )
        perm = (jnp.broadcast_to(rank_rowv, (C, C))
                == subl_id.astype(jnp.float32)).astype(jnp.float32)
        out_ref[...] = lax.dot_general(
            perm, proto, dimension_numbers=(((1,), (0,)), ((), ())),
            preferred_element_type=jnp.float32,
            precision=lax.Precision.HIGHEST,
        )


def _tc_finish(psums, tc_sums, cnt, fi, lab_sc, W, b2d):
    PG = 2 * NW // FIN_K
    return pl.pallas_call(
        _tc_body,
        grid=(FIN_K,),
        in_specs=[
            pl.BlockSpec((PG, C, D_IN), lambda k: (k, 0, 0)),
            pl.BlockSpec((C, D_IN), lambda k: (0, 0)),
            pl.BlockSpec((C, 128), lambda k: (0, 0)),
            pl.BlockSpec((C, 128), lambda k: (0, 0)),
            pl.BlockSpec((1, N), lambda k: (0, 0)),
            pl.BlockSpec((D_IN, D_OUT), lambda k: (0, 0)),
            pl.BlockSpec((1, D_OUT), lambda k: (0, 0)),
        ],
        out_specs=pl.BlockSpec((C, D_OUT), lambda k: (0, 0)),
        out_shape=jax.ShapeDtypeStruct((C, D_OUT), jnp.float32),
        scratch_shapes=[pltpu.VMEM((C, D_IN), jnp.float32)],
        compiler_params=pltpu.CompilerParams(
            dimension_semantics=("arbitrary",),
        ),
    )(psums, tc_sums, cnt, fi, lab_sc, W, b2d)


@jax.jit
def kernel(inputs, labels, W, b):
    labels_flat = labels.reshape(N)
    psums = _sc_segsum(inputs, labels_flat)
    tc_sums, cnt, fi = _tc_segsum(
        labels_flat.reshape(N // TCCHUNK, 1, TCCHUNK), inputs)
    return _tc_finish(psums, tc_sums, cnt, fi, labels_flat.reshape(1, N), W,
                      b.reshape(1, D_OUT))


# speedup vs baseline: 1.0068x; 1.0068x over previous
import functools

import jax
import jax.numpy as jnp
from jax import lax
from jax.experimental import pallas as pl
from jax.experimental.pallas import tpu as pltpu
from jax.experimental.pallas import tpu_sc as plsc

N, D_IN, D_OUT, C = 8192, 512, 512, 64
BIG = 2147483647

NC, NS = 2, 16
NSA = 16
NW = NC * NSA
F_SC = 2048
ROWS_SC_W = F_SC // NW
SCCHUNK = 32
NCHUNK = ROWS_SC_W // SCCHUNK
NV = D_IN // 16

TCCHUNK = 512
TC_K0 = F_SC // TCCHUNK
TC_K = (N - F_SC) // TCCHUNK
FIN_K = 2

_sc_mesh = plsc.VectorSubcoreMesh(core_axis_name="c", subcore_axis_name="s")


@functools.partial(
    pl.kernel,
    out_type=jax.ShapeDtypeStruct((2 * NW, C, D_IN), jnp.float32),
    mesh=_sc_mesh,
    scratch_types=[
        pltpu.VMEM((2, SCCHUNK, D_IN), jnp.float32),
        pltpu.VMEM((ROWS_SC_W + 16,), jnp.int32),
        pltpu.VMEM((C, D_IN), jnp.float32),
        pltpu.VMEM((C, D_IN), jnp.float32),
        pltpu.SemaphoreType.DMA,
        pltpu.SemaphoreType.DMA,
    ],
)
def _sc_segsum(x_hbm, lab_hbm, sums_hbm, xbuf, labv, acc0, acc1, sem0, sem1):
    c = lax.axis_index("c")
    s = lax.axis_index("s")
    wid = c * NSA + s
    base = wid * ROWS_SC_W

    @pl.when(s < NSA)
    def _worker():
        _sc_segsum_worker(x_hbm, lab_hbm, sums_hbm, xbuf, labv, acc0, acc1,
                          sem0, sem1, wid, base)


def _sc_segsum_worker(x_hbm, lab_hbm, sums_hbm, xbuf, labv, acc0, acc1,
                      sem0, sem1, wid, base):

    pltpu.sync_copy(lab_hbm.at[pl.ds(base, ROWS_SC_W)],
                    labv.at[pl.ds(0, ROWS_SC_W)])
    pltpu.async_copy(x_hbm.at[pl.ds(base, SCCHUNK)], xbuf.at[0], sem0)

    zv = jnp.zeros((16,), jnp.float32)

    def zero_row(r, carry):
        for jj in range(NV):
            acc0[r, pl.ds(jj * 16, 16)] = zv
            acc1[r, pl.ds(jj * 16, 16)] = zv
        return carry

    lax.fori_loop(0, C, zero_row, 0)

    def process(buf, chunk):
        def row_group(g, carry):
            labvec = labv[pl.ds(chunk * SCCHUNK + g * 8, 16)]
            LAG = 8
            for r8 in range(8):
                lab = labvec[r8]
                dst = acc0 if r8 % 2 == 0 else acc1
                row = g * 8 + r8
                vals = [None] * NV
                for jj in range(NV + LAG):
                    if jj < NV:
                        vals[jj] = xbuf[buf, row, pl.ds(jj * 16, 16)]
                    if jj >= LAG:
                        plsc.addupdate(
                            dst.at[lab, pl.ds((jj - LAG) * 16, 16)],
                            vals[jj - LAG])
            return carry

        lax.fori_loop(0, SCCHUNK // 8, row_group, 0)

    def pair_body(i2, carry):
        ca = 2 * i2
        pltpu.make_async_copy(
            x_hbm.at[pl.ds(base + ca * SCCHUNK, SCCHUNK)], xbuf.at[0], sem0
        ).wait()
        pltpu.async_copy(
            x_hbm.at[pl.ds(base + (ca + 1) * SCCHUNK, SCCHUNK)], xbuf.at[1],
            sem1)
        process(0, ca)
        pltpu.make_async_copy(
            x_hbm.at[pl.ds(base + (ca + 1) * SCCHUNK, SCCHUNK)], xbuf.at[1],
            sem1).wait()

        @pl.when(i2 + 1 < NCHUNK // 2)
        def _next():
            pltpu.async_copy(
                x_hbm.at[pl.ds(base + (ca + 2) * SCCHUNK, SCCHUNK)],
                xbuf.at[0], sem0)

        process(1, ca + 1)
        return carry

    lax.fori_loop(0, NCHUNK // 2, pair_body, 0)

    pltpu.sync_copy(acc0, sums_hbm.at[2 * wid])
    pltpu.sync_copy(acc1, sums_hbm.at[2 * wid + 1])


def _tc_segsum_body(labels_ref, x_ref, sums_ref, cnt_ref, fi_ref):
    k = pl.program_id(0)

    @pl.when(k == 0)
    def _init():
        sums_ref[...] = jnp.zeros_like(sums_ref)
        cnt_ref[...] = jnp.zeros_like(cnt_ref)
        fi_ref[...] = jnp.full_like(fi_ref, BIG)

    labels = labels_ref[0]
    lab_b = jnp.broadcast_to(labels, (C, TCCHUNK))
    class_ids = lax.broadcasted_iota(jnp.int32, (C, TCCHUNK), 0)
    onehot = lab_b == class_ids
    cnt = jnp.sum(onehot.astype(jnp.float32), axis=1, keepdims=True)
    cnt_ref[...] += jnp.broadcast_to(cnt, (C, 128))
    row_idx = ((k + TC_K0) * TCCHUNK
               + lax.broadcasted_iota(jnp.int32, (C, TCCHUNK), 1))
    masked = jnp.where(onehot, row_idx, BIG)
    fi_ref[...] = jnp.minimum(
        fi_ref[...],
        jnp.broadcast_to(jnp.min(masked, axis=1, keepdims=True), (C, 128)))

    x = x_ref[...]
    x_hi = x.astype(jnp.bfloat16)
    x_lo = (x - x_hi.astype(jnp.float32)).astype(jnp.bfloat16)
    oh_bf = onehot.astype(jnp.bfloat16)
    dn = (((1,), (0,)), ((), ()))
    sums_ref[...] += (
        lax.dot_general(oh_bf, x_hi, dimension_numbers=dn,
                        preferred_element_type=jnp.float32)
        + lax.dot_general(oh_bf, x_lo, dimension_numbers=dn,
                          preferred_element_type=jnp.float32)
    )


def _tc_segsum(labels3d, inputs):
    return pl.pallas_call(
        _tc_segsum_body,
        grid=(TC_K,),
        in_specs=[
            pl.BlockSpec((1, 1, TCCHUNK), lambda k: (k + TC_K0, 0, 0)),
            pl.BlockSpec((TCCHUNK, D_IN), lambda k: (k + TC_K0, 0)),
        ],
        out_specs=[
            pl.BlockSpec((C, D_IN), lambda k: (0, 0)),
            pl.BlockSpec((C, 128), lambda k: (0, 0)),
            pl.BlockSpec((C, 128), lambda k: (0, 0)),
        ],
        out_shape=[
            jax.ShapeDtypeStruct((C, D_IN), jnp.float32),
            jax.ShapeDtypeStruct((C, 128), jnp.float32),
            jax.ShapeDtypeStruct((C, 128), jnp.int32),
        ],
        compiler_params=pltpu.CompilerParams(
            dimension_semantics=("arbitrary",),
        ),
    )(labels3d, inputs)


def _tc_body(psums_ref, tcsums_ref, cnt_ref, fi_ref, labsc_ref, W_ref,
             b_ref, out_ref, acc_ref):
    k = pl.program_id(0)
    PG = 2 * NW // FIN_K

    @pl.when(k == 0)
    def _init():
        acc_ref[...] = tcsums_ref[...]

    sums = psums_ref[0]
    for t in range(1, PG):
        sums = sums + psums_ref[t]
    acc_ref[...] += sums

    @pl.when(k == FIN_K - 1)
    def _finish():
        labels = labsc_ref[:, :F_SC]
        lab_b = jnp.broadcast_to(labels, (C, F_SC))
        class_ids = lax.broadcasted_iota(jnp.int32, (C, F_SC), 0)
        onehot = lab_b == class_ids
        cnt0 = jnp.sum(onehot.astype(jnp.float32), axis=1, keepdims=True)
        row_idx = lax.broadcasted_iota(jnp.int32, (C, F_SC), 1)
        fi0 = jnp.min(jnp.where(onehot, row_idx, BIG), axis=1, keepdims=True)

        cnt_col = cnt_ref[:, :1] + cnt0
        fi_col_i = jnp.minimum(fi_ref[:, :1], fi0)

        sums_all = acc_ref[...]
        means = sums_all / jnp.broadcast_to(cnt_col, (C, D_IN))
        proto = lax.dot_general(
            means, W_ref[...], dimension_numbers=(((1,), (0,)), ((), ())),
            preferred_element_type=jnp.float32,
            precision=lax.Precision.HIGHEST,
        ) + b_ref[...]

        fi_col = fi_col_i.astype(jnp.float32)
        eye = (lax.broadcasted_iota(jnp.int32, (C, C), 0)
               == lax.broadcasted_iota(jnp.int32, (C, C), 1)
               ).astype(jnp.float32)
        fi_rowv = lax.dot_general(
            fi_col, eye, dimension_numbers=(((0,), (0,)), ((), ())),
            preferred_element_type=jnp.float32,
            precision=lax.Precision.HIGHEST,
        )
        fi_lanes = jnp.broadcast_to(fi_rowv, (C, C))
        fi_subl = jnp.broadcast_to(fi_col, (C, C))
        lane_id = lax.broadcasted_iota(jnp.int32, (C, C), 1)
        subl_id = lax.broadcasted_iota(jnp.int32, (C, C), 0)
        less = ((fi_lanes < fi_subl)
                | ((fi_lanes == fi_subl) & (lane_id < subl_id)))
        rank_col = jnp.sum(less.astype(jnp.float32), axis=1, keepdims=True)
        rank_rowv = lax.dot_general(
            rank_col, eye, dimension_numbers=(((0,), (0,)), ((), ())),
            preferred_element_type=jnp.float32,
            precision=lax.Precision.HIGHEST,
        )
        perm = (jnp.broadcast_to(rank_rowv, (C, C))
                == subl_id.astype(jnp.float32)).astype(jnp.float32)
        out_ref[...] = lax.dot_general(
            perm, proto, dimension_numbers=(((1,), (0,)), ((), ())),
            preferred_element_type=jnp.float32,
            precision=lax.Precision.HIGHEST,
        )


def _tc_finish(psums, tc_sums, cnt, fi, lab_sc, W, b2d):
    PG = 2 * NW // FIN_K
    return pl.pallas_call(
        _tc_body,
        grid=(FIN_K,),
        in_specs=[
            pl.BlockSpec((PG, C, D_IN), lambda k: (k, 0, 0)),
            pl.BlockSpec((C, D_IN), lambda k: (0, 0)),
            pl.BlockSpec((C, 128), lambda k: (0, 0)),
            pl.BlockSpec((C, 128), lambda k: (0, 0)),
            pl.BlockSpec((1, N), lambda k: (0, 0)),
            pl.BlockSpec((D_IN, D_OUT), lambda k: (0, 0)),
            pl.BlockSpec((1, D_OUT), lambda k: (0, 0)),
        ],
        out_specs=pl.BlockSpec((C, D_OUT), lambda k: (0, 0)),
        out_shape=jax.ShapeDtypeStruct((C, D_OUT), jnp.float32),
        scratch_shapes=[pltpu.VMEM((C, D_IN), jnp.float32)],
        compiler_params=pltpu.CompilerParams(
            dimension_semantics=("arbitrary",),
        ),
    )(psums, tc_sums, cnt, fi, lab_sc, W, b2d)


@jax.jit
def kernel(inputs, labels, W, b):
    labels_flat = labels.reshape(N)
    psums = _sc_segsum(inputs, labels_flat)
    tc_sums, cnt, fi = _tc_segsum(
        labels_flat.reshape(N // TCCHUNK, 1, TCCHUNK), inputs)
    return _tc_finish(psums, tc_sums, cnt, fi, labels_flat.reshape(1, N), W,
                      b.reshape(1, D_OUT))
